# baseline (speedup 1.0000x reference)
import functools

import jax
import jax.numpy as jnp
from jax import lax
from jax.experimental import pallas as pl
from jax.experimental.pallas import tpu as pltpu
from jax.experimental.pallas import tpu_sc as plsc

ROWS = 64
COLS = 32768
LANES = 16
NUM_CORES = 2
NUM_SUBCORES = 16
NUM_WORKERS = NUM_CORES * NUM_SUBCORES
ROWS_PER_WORKER = ROWS // NUM_WORKERS
NVREGS = COLS // LANES
GROUP_VREGS = 4
GROUP_COLS = GROUP_VREGS * LANES
NGROUPS = COLS // GROUP_COLS
BISECT_ITERS = 26

_MESH = plsc.VectorSubcoreMesh(core_axis_name="c", subcore_axis_name="s")

_GATHER_DNUMS = lax.GatherDimensionNumbers(
    offset_dims=(), collapsed_slice_dims=(0,), start_index_map=(0,)
)


def _shuffle(v, sh):
    idx = jnp.bitwise_xor(lax.iota(jnp.int32, LANES), sh)
    return lax.gather(
        v,
        idx[:, None],
        dimension_numbers=_GATHER_DNUMS,
        slice_sizes=(1,),
        mode=lax.GatherScatterMode.PROMISE_IN_BOUNDS,
    )


def _allmax(v):
    for sh in (8, 4, 2, 1):
        v = jnp.maximum(v, _shuffle(v, sh))
    return v


def _allsum(v):
    for sh in (8, 4, 2, 1):
        v = v + _shuffle(v, sh)
    return v


def _row_sparsemax(row_v, cand_v, gsum_v, gidx_v):
    f32 = jnp.float32
    lane_id = lax.iota(jnp.int32, LANES)

    for t in range(GROUP_VREGS):
        row_v[pl.ds(COLS + t * LANES, LANES)] = jnp.full((LANES,), -3.4e38, f32)

    @plsc.parallel_loop(0, NGROUPS, step=1, unroll=4,
                        carry=jnp.full((LANES,), -3.4e38, f32))
    def acc(g, acc_c):
        base = g * GROUP_COLS
        v0 = row_v[pl.ds(base, LANES)]
        v1 = row_v[pl.ds(base + LANES, LANES)]
        v2 = row_v[pl.ds(base + 2 * LANES, LANES)]
        v3 = row_v[pl.ds(base + 3 * LANES, LANES)]
        gmax = jnp.maximum(jnp.maximum(v0, v1), jnp.maximum(v2, v3))
        gsum_v[pl.ds(g * LANES, LANES)] = gmax
        return jnp.maximum(acc_c, gmax)

    maxv = _allmax(acc)
    thr = maxv - 1.0

    @plsc.parallel_loop(0, NGROUPS, step=1, unroll=4,
                        carry=jnp.zeros((LANES,), jnp.int32))
    def offg_vec(g, offg_c):
        s = gsum_v[pl.ds(g * LANES, LANES)]
        any_cand = plsc.all_reduce_population_count(s > thr) > 0
        m2 = jnp.logical_and(any_cand, lane_id == 0)
        plsc.store_scatter(gidx_v, [offg_c], jnp.full((LANES,), g, jnp.int32),
                           mask=m2)
        return offg_c + plsc.all_reduce_population_count(m2)

    nflag = offg_vec[0]
    gidx_v[pl.ds(nflag, LANES)] = jnp.full((LANES,), NGROUPS, jnp.int32)
    nouter = ((offg_vec + (LANES - 1)) // LANES)[0]

    def cp_body(o, off_c):
        gv = gidx_v[pl.ds(o * LANES, LANES)]
        off = off_c
        for k in range(LANES):
            base = gv[k] * GROUP_COLS
            for t in range(GROUP_VREGS):
                v = row_v[pl.ds(base + t * LANES, LANES)]
                m = v > thr
                c = plsc.cumsum(m.astype(jnp.int32))
                plsc.store_scatter(cand_v, [off + c - 1], v, mask=m)
                off = off + plsc.all_reduce_population_count(m)
        return off

    off_vec = lax.fori_loop(0, nouter, cp_body, jnp.zeros((LANES,), jnp.int32))
    nc = off_vec[0]
    cand_v[pl.ds(nc, LANES)] = thr
    nv = ((off_vec + (LANES - 1)) // LANES)[0]

    def bis_body(_, carry):
        lo, hi = carry
        mid = 0.5 * (lo + hi)

        def s_body(j, sacc):
            v = cand_v[pl.ds(j * LANES, LANES)]
            return sacc + jnp.maximum(v - mid, 0.0)

        sacc = lax.fori_loop(0, nv, s_body, jnp.zeros((LANES,), f32))
        pred = _allsum(sacc) >= 1.0
        return jnp.where(pred, mid, lo), jnp.where(pred, hi, mid)

    lo, _ = lax.fori_loop(0, BISECT_ITERS, bis_body, (thr, maxv))

    def rf_body(j, carry):
        sacc, cacc = carry
        v = cand_v[pl.ds(j * LANES, LANES)]
        m = v > lo
        return sacc + jnp.where(m, v, 0.0), cacc + m.astype(f32)

    sacc, cacc = lax.fori_loop(
        0, nv, rf_body, (jnp.zeros((LANES,), f32), jnp.zeros((LANES,), f32))
    )
    t_vec = (_allsum(sacc) - 1.0) / _allsum(cacc)

    @plsc.parallel_loop(0, NVREGS, step=8, unroll=2)
    def _(j):
        for k in range(8):
            v = row_v[pl.ds((j + k) * LANES, LANES)]
            row_v[pl.ds((j + k) * LANES, LANES)] = jnp.maximum(v - t_vec, 0.0)


@functools.partial(
    pl.kernel,
    out_type=jax.ShapeDtypeStruct((ROWS, COLS), jnp.float32),
    mesh=_MESH,
    scratch_types=[
        pltpu.VMEM((COLS + GROUP_COLS,), jnp.float32),
        pltpu.VMEM((COLS + GROUP_COLS,), jnp.float32),
        pltpu.VMEM((COLS + LANES,), jnp.float32),
        pltpu.VMEM((NGROUPS * LANES,), jnp.float32),
        pltpu.VMEM((NGROUPS + LANES,), jnp.int32),
        pltpu.SemaphoreType.DMA,
        pltpu.SemaphoreType.DMA,
        pltpu.SemaphoreType.DMA,
        pltpu.SemaphoreType.DMA,
    ],
    compiler_params=pltpu.CompilerParams(needs_layout_passes=False),
)
def _sparsemax_sc(x_hbm, o_hbm, row0_v, row1_v, cand_v, gsum_v, gidx_v,
                  sem_i0, sem_i1, sem_o0, sem_o1):
    wid = lax.axis_index("s") * NUM_CORES + lax.axis_index("c")
    r0 = wid * ROWS_PER_WORKER
    r1 = r0 + 1
    cp_i0 = pltpu.async_copy(x_hbm.at[r0], row0_v.at[pl.ds(0, COLS)], sem_i0)
    cp_i1 = pltpu.async_copy(x_hbm.at[r1], row1_v.at[pl.ds(0, COLS)], sem_i1)
    cp_i0.wait()
    _row_sparsemax(row0_v, cand_v, gsum_v, gidx_v)
    cp_o0 = pltpu.async_copy(row0_v.at[pl.ds(0, COLS)], o_hbm.at[r0], sem_o0)
    cp_i1.wait()
    _row_sparsemax(row1_v, cand_v, gsum_v, gidx_v)
    cp_o1 = pltpu.async_copy(row1_v.at[pl.ds(0, COLS)], o_hbm.at[r1], sem_o1)
    cp_o0.wait()
    cp_o1.wait()


def kernel(inputs):
    return _sparsemax_sc(inputs)


# speedup vs baseline: 1.1470x; 1.1470x over previous
import functools

import jax
import jax.numpy as jnp
from jax import lax
from jax.experimental import pallas as pl
from jax.experimental.pallas import tpu as pltpu
from jax.experimental.pallas import tpu_sc as plsc

ROWS = 64
COLS = 32768
LANES = 16
NUM_CORES = 2
NUM_SUBCORES = 16
NUM_WORKERS = NUM_CORES * NUM_SUBCORES
ROWS_PER_WORKER = ROWS // NUM_WORKERS
NVREGS = COLS // LANES
GROUP_VREGS = 4
GROUP_COLS = GROUP_VREGS * LANES
NGROUPS = COLS // GROUP_COLS
BISECT_ITERS = 26

_MESH = plsc.VectorSubcoreMesh(core_axis_name="c", subcore_axis_name="s")

_GATHER_DNUMS = lax.GatherDimensionNumbers(
    offset_dims=(), collapsed_slice_dims=(0,), start_index_map=(0,)
)


def _shuffle(v, sh):
    idx = jnp.bitwise_xor(lax.iota(jnp.int32, LANES), sh)
    return lax.gather(
        v,
        idx[:, None],
        dimension_numbers=_GATHER_DNUMS,
        slice_sizes=(1,),
        mode=lax.GatherScatterMode.PROMISE_IN_BOUNDS,
    )


def _allmax(v):
    for sh in (8, 4, 2, 1):
        v = jnp.maximum(v, _shuffle(v, sh))
    return v


def _allsum(v):
    for sh in (8, 4, 2, 1):
        v = v + _shuffle(v, sh)
    return v


def _row_sparsemax(row_v, cand_v):
    f32 = jnp.float32

    @plsc.parallel_loop(0, NVREGS, step=8, unroll=2,
                        carry=jnp.full((LANES,), -3.4e38, f32))
    def acc(j, acc_c):
        vs = [row_v[pl.ds((j + k) * LANES, LANES)] for k in range(8)]
        m01 = jnp.maximum(vs[0], vs[1])
        m23 = jnp.maximum(vs[2], vs[3])
        m45 = jnp.maximum(vs[4], vs[5])
        m67 = jnp.maximum(vs[6], vs[7])
        m = jnp.maximum(jnp.maximum(m01, m23), jnp.maximum(m45, m67))
        return jnp.maximum(acc_c, m)

    maxv = _allmax(acc)
    thr = maxv - 1.0

    @plsc.parallel_loop(0, NVREGS, step=1, unroll=8,
                        carry=jnp.zeros((LANES,), jnp.int32))
    def off_vec(j, off_c):
        v = row_v[pl.ds(j * LANES, LANES)]
        m = v > thr
        c = plsc.cumsum(m.astype(jnp.int32))
        plsc.store_scatter(cand_v, [off_c + c - 1], v, mask=m)
        return off_c + plsc.all_reduce_population_count(m)

    nc = off_vec[0]
    cand_v[pl.ds(nc, LANES)] = thr
    nv = ((off_vec + (LANES - 1)) // LANES)[0]

    def bis_body(_, carry):
        lo, hi = carry
        mid = 0.5 * (lo + hi)

        def s_body(j, sacc):
            v = cand_v[pl.ds(j * LANES, LANES)]
            return sacc + jnp.maximum(v - mid, 0.0)

        sacc = lax.fori_loop(0, nv, s_body, jnp.zeros((LANES,), f32))
        pred = _allsum(sacc) >= 1.0
        return jnp.where(pred, mid, lo), jnp.where(pred, hi, mid)

    lo, _ = lax.fori_loop(0, BISECT_ITERS, bis_body, (thr, maxv))

    def rf_body(j, carry):
        sacc, cacc = carry
        v = cand_v[pl.ds(j * LANES, LANES)]
        m = v > lo
        return sacc + jnp.where(m, v, 0.0), cacc + m.astype(f32)

    sacc, cacc = lax.fori_loop(
        0, nv, rf_body, (jnp.zeros((LANES,), f32), jnp.zeros((LANES,), f32))
    )
    t_vec = (_allsum(sacc) - 1.0) / _allsum(cacc)

    @plsc.parallel_loop(0, NVREGS, step=8, unroll=2)
    def _(j):
        for k in range(8):
            v = row_v[pl.ds((j + k) * LANES, LANES)]
            row_v[pl.ds((j + k) * LANES, LANES)] = jnp.maximum(v - t_vec, 0.0)


@functools.partial(
    pl.kernel,
    out_type=jax.ShapeDtypeStruct((ROWS, COLS), jnp.float32),
    mesh=_MESH,
    scratch_types=[
        pltpu.VMEM((COLS,), jnp.float32),
        pltpu.VMEM((COLS,), jnp.float32),
        pltpu.VMEM((COLS + LANES,), jnp.float32),
        pltpu.SemaphoreType.DMA,
        pltpu.SemaphoreType.DMA,
        pltpu.SemaphoreType.DMA,
        pltpu.SemaphoreType.DMA,
    ],
    compiler_params=pltpu.CompilerParams(needs_layout_passes=False),
)
def _sparsemax_sc(x_hbm, o_hbm, row0_v, row1_v, cand_v,
                  sem_i0, sem_i1, sem_o0, sem_o1):
    wid = lax.axis_index("s") * NUM_CORES + lax.axis_index("c")
    r0 = wid * ROWS_PER_WORKER
    r1 = r0 + 1
    cp_i0 = pltpu.async_copy(x_hbm.at[r0], row0_v, sem_i0)
    cp_i1 = pltpu.async_copy(x_hbm.at[r1], row1_v, sem_i1)
    cp_i0.wait()
    _row_sparsemax(row0_v, cand_v)
    cp_o0 = pltpu.async_copy(row0_v, o_hbm.at[r0], sem_o0)
    cp_i1.wait()
    _row_sparsemax(row1_v, cand_v)
    cp_o1 = pltpu.async_copy(row1_v, o_hbm.at[r1], sem_o1)
    cp_o0.wait()
    cp_o1.wait()


def kernel(inputs):
    return _sparsemax_sc(inputs)


# speedup vs baseline: 1.1874x; 1.0352x over previous
import functools

import jax
import jax.numpy as jnp
from jax import lax
from jax.experimental import pallas as pl
from jax.experimental.pallas import tpu as pltpu
from jax.experimental.pallas import tpu_sc as plsc

ROWS = 64
COLS = 32768
LANES = 16
NUM_CORES = 2
NUM_SUBCORES = 16
NUM_WORKERS = NUM_CORES * NUM_SUBCORES
ROWS_PER_WORKER = ROWS // NUM_WORKERS
NVREGS = COLS // LANES
GROUP_VREGS = 4
GROUP_COLS = GROUP_VREGS * LANES
NGROUPS = COLS // GROUP_COLS
BISECT_ITERS = 26

_MESH = plsc.VectorSubcoreMesh(core_axis_name="c", subcore_axis_name="s")

_GATHER_DNUMS = lax.GatherDimensionNumbers(
    offset_dims=(), collapsed_slice_dims=(0,), start_index_map=(0,)
)


def _shuffle(v, sh):
    idx = jnp.bitwise_xor(lax.iota(jnp.int32, LANES), sh)
    return lax.gather(
        v,
        idx[:, None],
        dimension_numbers=_GATHER_DNUMS,
        slice_sizes=(1,),
        mode=lax.GatherScatterMode.PROMISE_IN_BOUNDS,
    )


def _allmax(v):
    for sh in (8, 4, 2, 1):
        v = jnp.maximum(v, _shuffle(v, sh))
    return v


def _allsum(v):
    for sh in (8, 4, 2, 1):
        v = v + _shuffle(v, sh)
    return v


def _max_pass(row_v, lo_vreg, hi_vreg, acc_init):
    @plsc.parallel_loop(lo_vreg, hi_vreg, step=8, unroll=2, carry=acc_init)
    def acc(j, acc_c):
        vs = [row_v[pl.ds((j + k) * LANES, LANES)] for k in range(8)]
        m01 = jnp.maximum(vs[0], vs[1])
        m23 = jnp.maximum(vs[2], vs[3])
        m45 = jnp.maximum(vs[4], vs[5])
        m67 = jnp.maximum(vs[6], vs[7])
        m = jnp.maximum(jnp.maximum(m01, m23), jnp.maximum(m45, m67))
        return jnp.maximum(acc_c, m)

    return acc


def _row_tau(row_v, cand_v, acc):
    f32 = jnp.float32
    maxv = _allmax(acc)
    thr = maxv - 1.0

    @plsc.parallel_loop(0, NVREGS, step=1, unroll=8,
                        carry=jnp.zeros((LANES,), jnp.int32))
    def off_vec(j, off_c):
        v = row_v[pl.ds(j * LANES, LANES)]
        m = v > thr
        c = plsc.cumsum(m.astype(jnp.int32))
        plsc.store_scatter(cand_v, [off_c + c - 1], v, mask=m)
        return off_c + plsc.all_reduce_population_count(m)

    nc = off_vec[0]
    cand_v[pl.ds(nc, LANES)] = thr
    nv = ((off_vec + (LANES - 1)) // LANES)[0]

    def bis_body(_, carry):
        lo, hi = carry
        mid = 0.5 * (lo + hi)

        def s_body(j, sacc):
            v = cand_v[pl.ds(j * LANES, LANES)]
            return sacc + jnp.maximum(v - mid, 0.0)

        sacc = lax.fori_loop(0, nv, s_body, jnp.zeros((LANES,), f32))
        pred = _allsum(sacc) >= 1.0
        return jnp.where(pred, mid, lo), jnp.where(pred, hi, mid)

    lo, _ = lax.fori_loop(0, BISECT_ITERS, bis_body, (thr, maxv))

    def rf_body(j, carry):
        sacc, cacc = carry
        v = cand_v[pl.ds(j * LANES, LANES)]
        m = v > lo
        return sacc + jnp.where(m, v, 0.0), cacc + m.astype(f32)

    sacc, cacc = lax.fori_loop(
        0, nv, rf_body, (jnp.zeros((LANES,), f32), jnp.zeros((LANES,), f32))
    )
    t_vec = (_allsum(sacc) - 1.0) / _allsum(cacc)
    return t_vec


def _out_pass(row_v, t_vec, lo_vreg, hi_vreg):
    @plsc.parallel_loop(lo_vreg, hi_vreg, step=8, unroll=2)
    def _(j):
        for k in range(8):
            v = row_v[pl.ds((j + k) * LANES, LANES)]
            row_v[pl.ds((j + k) * LANES, LANES)] = jnp.maximum(v - t_vec, 0.0)


NCHUNKS = 4
CHUNK_COLS = COLS // NCHUNKS
CHUNK_VREGS = NVREGS // NCHUNKS


@functools.partial(
    pl.kernel,
    out_type=jax.ShapeDtypeStruct((ROWS, COLS), jnp.float32),
    mesh=_MESH,
    scratch_types=[
        pltpu.VMEM((COLS,), jnp.float32),
        pltpu.VMEM((COLS,), jnp.float32),
        pltpu.VMEM((COLS + LANES,), jnp.float32),
        [pltpu.SemaphoreType.DMA] * NCHUNKS,
        pltpu.SemaphoreType.DMA,
        pltpu.SemaphoreType.DMA,
    ],
    compiler_params=pltpu.CompilerParams(needs_layout_passes=False),
)
def _sparsemax_sc(x_hbm, o_hbm, row0_v, row1_v, cand_v,
                  sems_c, sem_i1, sem_o0):
    wid = lax.axis_index("s") * NUM_CORES + lax.axis_index("c")
    r0 = wid * ROWS_PER_WORKER
    r1 = r0 + 1
    cps0 = [
        pltpu.async_copy(
            x_hbm.at[r0, pl.ds(k * CHUNK_COLS, CHUNK_COLS)],
            row0_v.at[pl.ds(k * CHUNK_COLS, CHUNK_COLS)],
            sems_c[k],
        )
        for k in range(NCHUNKS)
    ]
    cp_i1 = pltpu.async_copy(x_hbm.at[r1], row1_v, sem_i1)
    acc = jnp.full((LANES,), -3.4e38, jnp.float32)
    for k in range(NCHUNKS):
        cps0[k].wait()
        acc = _max_pass(row0_v, k * CHUNK_VREGS, (k + 1) * CHUNK_VREGS, acc)
    t0 = _row_tau(row0_v, cand_v, acc)
    _out_pass(row0_v, t0, 0, NVREGS)
    cp_o0 = pltpu.async_copy(row0_v, o_hbm.at[r0], sem_o0)
    cp_i1.wait()
    acc1 = _max_pass(row1_v, 0, NVREGS, jnp.full((LANES,), -3.4e38, jnp.float32))
    t1 = _row_tau(row1_v, cand_v, acc1)
    cps1 = []
    for k in range(NCHUNKS):
        _out_pass(row1_v, t1, k * CHUNK_VREGS, (k + 1) * CHUNK_VREGS)
        cps1.append(
            pltpu.async_copy(
                row1_v.at[pl.ds(k * CHUNK_COLS, CHUNK_COLS)],
                o_hbm.at[r1, pl.ds(k * CHUNK_COLS, CHUNK_COLS)],
                sems_c[k],
            )
        )
    cp_o0.wait()
    for cp in cps1:
        cp.wait()


def kernel(inputs):
    return _sparsemax_sc(inputs)


# speedup vs baseline: 1.1918x; 1.0037x over previous
import functools

import jax
import jax.numpy as jnp
from jax import lax
from jax.experimental import pallas as pl
from jax.experimental.pallas import tpu as pltpu
from jax.experimental.pallas import tpu_sc as plsc

ROWS = 64
COLS = 32768
LANES = 16
NUM_CORES = 2
NUM_SUBCORES = 16
NUM_WORKERS = NUM_CORES * NUM_SUBCORES
ROWS_PER_WORKER = ROWS // NUM_WORKERS
NVREGS = COLS // LANES
GROUP_VREGS = 4
GROUP_COLS = GROUP_VREGS * LANES
NGROUPS = COLS // GROUP_COLS
BISECT_ITERS = 26

_MESH = plsc.VectorSubcoreMesh(core_axis_name="c", subcore_axis_name="s")

_GATHER_DNUMS = lax.GatherDimensionNumbers(
    offset_dims=(), collapsed_slice_dims=(0,), start_index_map=(0,)
)


def _shuffle(v, sh):
    idx = jnp.bitwise_xor(lax.iota(jnp.int32, LANES), sh)
    return lax.gather(
        v,
        idx[:, None],
        dimension_numbers=_GATHER_DNUMS,
        slice_sizes=(1,),
        mode=lax.GatherScatterMode.PROMISE_IN_BOUNDS,
    )


def _allmax(v):
    for sh in (8, 4, 2, 1):
        v = jnp.maximum(v, _shuffle(v, sh))
    return v


def _allsum(v):
    for sh in (8, 4, 2, 1):
        v = v + _shuffle(v, sh)
    return v


def _max_pass(row_v, lo_vreg, hi_vreg, acc_init):
    @plsc.parallel_loop(lo_vreg, hi_vreg, step=8, unroll=2, carry=acc_init)
    def acc(j, acc_c):
        vs = [row_v[pl.ds((j + k) * LANES, LANES)] for k in range(8)]
        m01 = jnp.maximum(vs[0], vs[1])
        m23 = jnp.maximum(vs[2], vs[3])
        m45 = jnp.maximum(vs[4], vs[5])
        m67 = jnp.maximum(vs[6], vs[7])
        m = jnp.maximum(jnp.maximum(m01, m23), jnp.maximum(m45, m67))
        return jnp.maximum(acc_c, m)

    return acc


def _row_tau(row_v, cand_v, acc):
    f32 = jnp.float32
    maxv = _allmax(acc)
    thr = maxv - 1.0

    @plsc.parallel_loop(0, NVREGS, step=1, unroll=8,
                        carry=jnp.zeros((LANES,), jnp.int32))
    def off_vec(j, off_c):
        v = row_v[pl.ds(j * LANES, LANES)]
        m = v > thr
        c = plsc.cumsum(m.astype(jnp.int32))
        plsc.store_scatter(cand_v, [off_c + c - 1], v, mask=m)
        return off_c + plsc.all_reduce_population_count(m)

    nc = off_vec[0]
    cand_v[pl.ds(nc, LANES)] = thr
    nv = ((off_vec + (LANES - 1)) // LANES)[0]

    def bis_body(_, carry):
        lo, hi = carry
        mid = 0.5 * (lo + hi)

        def s_body(j, sacc):
            v = cand_v[pl.ds(j * LANES, LANES)]
            return sacc + jnp.maximum(v - mid, 0.0)

        sacc = lax.fori_loop(0, nv, s_body, jnp.zeros((LANES,), f32))
        pred = _allsum(sacc) >= 1.0
        return jnp.where(pred, mid, lo), jnp.where(pred, hi, mid)

    lo, _ = lax.fori_loop(0, BISECT_ITERS, bis_body, (thr, maxv))

    def rf_body(j, carry):
        sacc, cacc = carry
        v = cand_v[pl.ds(j * LANES, LANES)]
        m = v > lo
        return sacc + jnp.where(m, v, 0.0), cacc + m.astype(f32)

    sacc, cacc = lax.fori_loop(
        0, nv, rf_body, (jnp.zeros((LANES,), f32), jnp.zeros((LANES,), f32))
    )
    t_vec = (_allsum(sacc) - 1.0) / _allsum(cacc)
    return t_vec


def _out_pass(row_v, t_vec, lo_vreg, hi_vreg):
    @plsc.parallel_loop(lo_vreg, hi_vreg, step=8, unroll=2)
    def _(j):
        for k in range(8):
            v = row_v[pl.ds((j + k) * LANES, LANES)]
            row_v[pl.ds((j + k) * LANES, LANES)] = jnp.maximum(v - t_vec, 0.0)


NCHUNKS = 4
CHUNK_COLS = COLS // NCHUNKS
CHUNK_VREGS = NVREGS // NCHUNKS


@functools.partial(
    pl.kernel,
    out_type=jax.ShapeDtypeStruct((ROWS, COLS), jnp.float32),
    mesh=_MESH,
    scratch_types=[
        pltpu.VMEM((COLS,), jnp.float32),
        pltpu.VMEM((COLS,), jnp.float32),
        pltpu.VMEM((COLS + LANES,), jnp.float32),
        [pltpu.SemaphoreType.DMA] * NCHUNKS,
        pltpu.SemaphoreType.DMA,
        pltpu.SemaphoreType.DMA,
    ],
    compiler_params=pltpu.CompilerParams(needs_layout_passes=False),
)
def _sparsemax_sc(x_hbm, o_hbm, row0_v, row1_v, cand_v,
                  sems_c, sem_i1, sem_o0):
    wid = lax.axis_index("s") * NUM_CORES + lax.axis_index("c")
    r0 = wid * ROWS_PER_WORKER
    r1 = r0 + 1
    cps0 = [
        pltpu.async_copy(
            x_hbm.at[r0, pl.ds(k * CHUNK_COLS, CHUNK_COLS)],
            row0_v.at[pl.ds(k * CHUNK_COLS, CHUNK_COLS)],
            sems_c[k],
        )
        for k in range(NCHUNKS)
    ]
    cp_i1 = pltpu.async_copy(x_hbm.at[r1], row1_v, sem_i1)
    acc = jnp.full((LANES,), -3.4e38, jnp.float32)
    with jax.named_scope("max0"):
        for k in range(NCHUNKS):
            cps0[k].wait()
            acc = _max_pass(row0_v, k * CHUNK_VREGS, (k + 1) * CHUNK_VREGS, acc)
    with jax.named_scope("tau0"):
        t0 = _row_tau(row0_v, cand_v, acc)
    with jax.named_scope("out0"):
        _out_pass(row0_v, t0, 0, NVREGS)
    cp_o0 = pltpu.async_copy(row0_v, o_hbm.at[r0], sem_o0)
    with jax.named_scope("wait_i1"):
        cp_i1.wait()
    with jax.named_scope("max1"):
        acc1 = _max_pass(row1_v, 0, NVREGS,
                         jnp.full((LANES,), -3.4e38, jnp.float32))
    with jax.named_scope("tau1"):
        t1 = _row_tau(row1_v, cand_v, acc1)
    cps1 = []
    for k in range(NCHUNKS):
        _out_pass(row1_v, t1, k * CHUNK_VREGS, (k + 1) * CHUNK_VREGS)
        cps1.append(
            pltpu.async_copy(
                row1_v.at[pl.ds(k * CHUNK_COLS, CHUNK_COLS)],
                o_hbm.at[r1, pl.ds(k * CHUNK_COLS, CHUNK_COLS)],
                sems_c[k],
            )
        )
    cp_o0.wait()
    for cp in cps1:
        cp.wait()


def kernel(inputs):
    return _sparsemax_sc(inputs)
